# Initial kernel scaffold; baseline (speedup 1.0000x reference)
import jax
import jax.numpy as jnp
from jax.experimental import pallas as pl


def kernel(unified_embedding, logits, gt, is_train, dataset_ids, features, ex, ex2, count):
    raise NotImplementedError("write your pallas kernel here")


# speedup vs baseline: 1.2821x; 1.2821x over previous
import jax
import jax.numpy as jnp
from jax import lax
from jax.experimental import pallas as pl
from jax.experimental.pallas import tpu as pltpu

_NB = 2048


def _body(is_train_ref, lg_ref, emb_ref, feat_ref, ex_ref, ex2_ref, cnt_in_ref,
          nf_ref, nex_ref, nex2_ref, ncnt_ref,
          sum_ref, sq_ref, cnt_ref):
    step = pl.program_id(0)
    nsteps = pl.num_programs(0)

    @pl.when(step == 0)
    def _init():
        sum_ref[...] = jnp.zeros_like(sum_ref)
        sq_ref[...] = jnp.zeros_like(sq_ref)
        cnt_ref[...] = jnp.zeros_like(cnt_ref)

    lg = lg_ref[0]
    e = emb_ref[0]
    L = lg.shape[0]
    sem = jnp.argmax(lg, axis=0)
    oh = (lax.broadcasted_iota(jnp.int32, (L, _NB), 0) == sem[None, :]
          ).astype(jnp.float32)
    dn = (((1,), (1,)), ((), ()))
    sum_ref[...] += lax.dot_general(oh, e, dn,
                                    preferred_element_type=jnp.float32)
    sq_ref[...] += lax.dot_general(oh, e * e, dn,
                                   preferred_element_type=jnp.float32)
    cnt_ref[...] += jnp.sum(oh, axis=1, keepdims=True)

    @pl.when(step == nsteps - 1)
    def _fin():
        train = (is_train_ref[0] != 0).astype(jnp.float32)
        s = sum_ref[...]
        q = sq_ref[...]
        c_new = cnt_ref[...]
        c_old = cnt_in_ref[...]
        feat = feat_ref[...]
        upd_f = (feat * c_old + s) / (c_old + c_new + 1e-8)
        nf_ref[...] = train * upd_f + (1.0 - train) * feat
        nex_ref[...] = ex_ref[...] + train * s
        nex2_ref[...] = ex2_ref[...] + train * q
        ncnt_ref[...] = c_old + train * c_new


def kernel(unified_embedding, logits, gt, is_train, dataset_ids, features, ex,
           ex2, count):
    B, C, H, W = unified_embedding.shape
    L = logits.shape[1]
    N = H * W
    emb = unified_embedding.reshape(B * 1, C, N).reshape(B, C, N)
    lg = logits.reshape(B, L, N)
    nj = N // _NB
    grid = (B * nj,)

    def lg_map(i):
        return (i // nj, 0, i * 0 + (i % nj))

    full = lambda i: (0, 0)
    it = jnp.asarray(is_train, jnp.int32).reshape(1)

    out = pl.pallas_call(
        _body,
        grid=grid,
        in_specs=[
            pl.BlockSpec(memory_space=pltpu.SMEM),
            pl.BlockSpec((1, L, _NB), lg_map),
            pl.BlockSpec((1, C, _NB), lg_map),
            pl.BlockSpec((L, C), full),
            pl.BlockSpec((L, C), full),
            pl.BlockSpec((L, C), full),
            pl.BlockSpec((L, 1), full),
        ],
        out_specs=[
            pl.BlockSpec((L, C), full),
            pl.BlockSpec((L, C), full),
            pl.BlockSpec((L, C), full),
            pl.BlockSpec((L, 1), full),
        ],
        out_shape=[
            jax.ShapeDtypeStruct((L, C), jnp.float32),
            jax.ShapeDtypeStruct((L, C), jnp.float32),
            jax.ShapeDtypeStruct((L, C), jnp.float32),
            jax.ShapeDtypeStruct((L, 1), jnp.float32),
        ],
        scratch_shapes=[
            pltpu.VMEM((L, C), jnp.float32),
            pltpu.VMEM((L, C), jnp.float32),
            pltpu.VMEM((L, 1), jnp.float32),
        ],
    )(it, lg, emb, features, ex, ex2, count.reshape(L, 1))

    new_features, new_ex, new_ex2, new_count = out
    acc_loss = jnp.zeros((), jnp.float32)
    return (acc_loss, new_features, new_ex, new_ex2, new_count.reshape(L))


# speedup vs baseline: 1.3251x; 1.0335x over previous
import jax
import jax.numpy as jnp
from jax import lax
from jax.experimental import pallas as pl
from jax.experimental.pallas import tpu as pltpu

_NB = 4096


def _body(is_train_ref, lg_ref, emb_ref, feat_ref, ex_ref, ex2_ref, cnt_in_ref,
          nf_ref, nex_ref, nex2_ref, ncnt_ref,
          sum_ref, sq_ref, cnt_ref):
    step = pl.program_id(0)
    nsteps = pl.num_programs(0)

    @pl.when(step == 0)
    def _init():
        sum_ref[...] = jnp.zeros_like(sum_ref)
        sq_ref[...] = jnp.zeros_like(sq_ref)
        cnt_ref[...] = jnp.zeros_like(cnt_ref)

    lg = lg_ref[0]
    e = emb_ref[0]
    L = lg.shape[0]
    sem = jnp.argmax(lg, axis=0)
    oh = (lax.broadcasted_iota(jnp.int32, (L, _NB), 0) == sem[None, :]
          ).astype(jnp.float32)
    dn = (((1,), (1,)), ((), ()))
    sum_ref[...] += lax.dot_general(oh, e, dn,
                                    preferred_element_type=jnp.float32)
    sq_ref[...] += lax.dot_general(oh, e * e, dn,
                                   preferred_element_type=jnp.float32)
    cnt_ref[...] += jnp.sum(oh, axis=1, keepdims=True)

    @pl.when(step == nsteps - 1)
    def _fin():
        train = (is_train_ref[0] != 0).astype(jnp.float32)
        s = sum_ref[...]
        q = sq_ref[...]
        c_new = cnt_ref[...]
        c_old = cnt_in_ref[...]
        feat = feat_ref[...]
        upd_f = (feat * c_old + s) / (c_old + c_new + 1e-8)
        nf_ref[...] = train * upd_f + (1.0 - train) * feat
        nex_ref[...] = ex_ref[...] + train * s
        nex2_ref[...] = ex2_ref[...] + train * q
        ncnt_ref[...] = c_old + train * c_new


def kernel(unified_embedding, logits, gt, is_train, dataset_ids, features, ex,
           ex2, count):
    B, C, H, W = unified_embedding.shape
    L = logits.shape[1]
    N = H * W
    emb = unified_embedding.reshape(B * 1, C, N).reshape(B, C, N)
    lg = logits.reshape(B, L, N)
    nj = N // _NB
    grid = (B * nj,)

    def lg_map(i):
        return (i // nj, 0, i * 0 + (i % nj))

    full = lambda i: (0, 0)
    it = jnp.asarray(is_train, jnp.int32).reshape(1)

    out = pl.pallas_call(
        _body,
        grid=grid,
        in_specs=[
            pl.BlockSpec(memory_space=pltpu.SMEM),
            pl.BlockSpec((1, L, _NB), lg_map),
            pl.BlockSpec((1, C, _NB), lg_map),
            pl.BlockSpec((L, C), full),
            pl.BlockSpec((L, C), full),
            pl.BlockSpec((L, C), full),
            pl.BlockSpec((L, 1), full),
        ],
        out_specs=[
            pl.BlockSpec((L, C), full),
            pl.BlockSpec((L, C), full),
            pl.BlockSpec((L, C), full),
            pl.BlockSpec((L, 1), full),
        ],
        out_shape=[
            jax.ShapeDtypeStruct((L, C), jnp.float32),
            jax.ShapeDtypeStruct((L, C), jnp.float32),
            jax.ShapeDtypeStruct((L, C), jnp.float32),
            jax.ShapeDtypeStruct((L, 1), jnp.float32),
        ],
        scratch_shapes=[
            pltpu.VMEM((L, C), jnp.float32),
            pltpu.VMEM((L, C), jnp.float32),
            pltpu.VMEM((L, 1), jnp.float32),
        ],
    )(it, lg, emb, features, ex, ex2, count.reshape(L, 1))

    new_features, new_ex, new_ex2, new_count = out
    acc_loss = jnp.zeros((), jnp.float32)
    return (acc_loss, new_features, new_ex, new_ex2, new_count.reshape(L))
